# Initial kernel scaffold; baseline (speedup 1.0000x reference)
import functools

import jax
import jax.numpy as jnp
from jax import lax
from jax.experimental import pallas as pl
from jax.experimental.pallas import tpu as pltpu
from jax.experimental.pallas import tpu_sc as plsc

_VOCAB = 100000
_D = 128
_S = 512
_B = 1024
_EPS = 1e-5

_NC = 2
_NS = 16
_NW = _NC * _NS
_N_TOK = _B * _S
_TOK_PER_W = _N_TOK // _NW
_CHUNK = 256
_N_CHUNK = _TOK_PER_W // _CHUNK
_GRP = _CHUNK // 16


def _rsqrt(x):
    i = plsc.bitcast(x, jnp.int32)
    i = jnp.int32(0x5F3759DF) - lax.shift_right_logical(i, 1)
    y = plsc.bitcast(i, jnp.float32)
    for _ in range(3):
        y = y * (1.5 - 0.5 * x * y * y)
    return y


def _body(ids_hbm, tt_hbm, wemb_hbm, pos_hbm, te_hbm, g_hbm, b_hbm, out_hbm,
          idx_v, tok_v, rows_v, pos_v, te_v, g_v, b_v, xT_v, sem):
    wid = lax.axis_index("s") * _NC + lax.axis_index("c")
    wbase = wid * _TOK_PER_W

    pltpu.sync_copy(pos_hbm, pos_v)
    pltpu.sync_copy(te_hbm, te_v)
    pltpu.sync_copy(g_hbm, g_v)
    pltpu.sync_copy(b_hbm, b_v)

    lane = lax.iota(jnp.int32, (16,))

    @pl.loop(0, _N_CHUNK)
    def _chunk(c):
        base = wbase + c * _CHUNK
        pltpu.sync_copy(ids_hbm.at[pl.ds(base, _CHUNK)], idx_v)
        pltpu.sync_copy(tt_hbm.at[pl.ds(base, _CHUNK)], tok_v)
        pltpu.async_copy(wemb_hbm.at[idx_v], rows_v, sem).wait()
        s0 = lax.rem(c * _CHUNK, _S)

        @pl.loop(0, _GRP)
        def _group(g):
            rowvec = g * 16 + lane
            posvec = s0 + rowvec
            tvec = tok_v[pl.ds(g * 16, 16)]

            def _pass1(j, carry):
                s, ss = carry
                jj = jnp.full((16,), j, jnp.int32)
                x = (plsc.load_gather(rows_v, [rowvec, jj])
                     + plsc.load_gather(pos_v, [posvec, jj])
                     + plsc.load_gather(te_v, [tvec, jj]))
                xT_v[pl.ds(j * 16, 16)] = x
                return s + x, ss + x * x

            zeros = jnp.zeros((16,), jnp.float32)
            sm, smsq = pl.loop(0, _D, init_carry=(zeros, zeros), unroll=4)(_pass1)

            mean = sm * (1.0 / _D)
            var = smsq * (1.0 / _D) - mean * mean
            rstd = _rsqrt(var + _EPS)

            @pl.loop(0, _D, unroll=4)
            def _pass2(j):
                jj = jnp.full((16,), j, jnp.int32)
                x = xT_v[pl.ds(j * 16, 16)]
                gj = plsc.load_gather(g_v, [jj])
                bj = plsc.load_gather(b_v, [jj])
                y = (x - mean) * rstd * gj + bj
                plsc.store_scatter(rows_v, [rowvec, jj], y)

        pltpu.sync_copy(rows_v, out_hbm.at[pl.ds(base, _CHUNK)])


@jax.jit
def kernel(input_ids, token_type_ids, word_emb, pos_emb, tok_type_emb, gamma,
           beta):
    ids = input_ids.reshape(_N_TOK)
    tts = token_type_ids.reshape(_N_TOK)
    mesh = plsc.VectorSubcoreMesh(core_axis_name="c", subcore_axis_name="s")
    run = functools.partial(
        pl.kernel,
        out_type=jax.ShapeDtypeStruct((_N_TOK, _D), jnp.float32),
        mesh=mesh,
        scratch_types=[
            pltpu.VMEM((_CHUNK,), jnp.int32),
            pltpu.VMEM((_CHUNK,), jnp.int32),
            pltpu.VMEM((_CHUNK, _D), jnp.float32),
            pltpu.VMEM((_S, _D), jnp.float32),
            pltpu.VMEM((2, _D), jnp.float32),
            pltpu.VMEM((_D,), jnp.float32),
            pltpu.VMEM((_D,), jnp.float32),
            pltpu.VMEM((16 * _D,), jnp.float32),
            pltpu.SemaphoreType.DMA,
        ],
    )(_body)
    return run(ids, tts, word_emb, pos_emb, tok_type_emb, gamma, beta)


# speedup vs baseline: 3.2755x; 3.2755x over previous
import functools

import jax
import jax.numpy as jnp
from jax import lax
from jax.experimental import pallas as pl
from jax.experimental.pallas import tpu as pltpu
from jax.experimental.pallas import tpu_sc as plsc

_VOCAB = 100000
_D = 128
_S = 512
_B = 1024
_EPS = 1e-5

_NC = 2
_NS = 16
_NW = _NC * _NS
_N_TOK = _B * _S
_TOK_PER_W = _N_TOK // _NW
_CHUNK = 256
_N_CHUNK = _TOK_PER_W // _CHUNK
_NK = _D // 16


def _rsqrt(x):
    i = plsc.bitcast(x, jnp.int32)
    i = jnp.int32(0x5F3759DF) - lax.shift_right_logical(i, 1)
    y = plsc.bitcast(i, jnp.float32)
    for _ in range(3):
        y = y * (1.5 - 0.5 * x * y * y)
    return y


def _body(ids_hbm, tt_hbm, wemb_hbm, pos_hbm, te_hbm, g_hbm, b_hbm, out_hbm,
          idx_v, tok_v, rows_v, pos_v, te_v, gb_v, sem):
    wid = lax.axis_index("s") * _NC + lax.axis_index("c")
    wbase = wid * _TOK_PER_W

    pltpu.sync_copy(pos_hbm, pos_v)
    pltpu.sync_copy(te_hbm, te_v)
    pltpu.sync_copy(g_hbm, gb_v.at[pl.ds(0, _D)])
    pltpu.sync_copy(b_hbm, gb_v.at[pl.ds(_D, _D)])

    te0 = [te_v[pl.ds(16 * k, 16)] for k in range(_NK)]
    dte = [te_v[pl.ds(_D + 16 * k, 16)] - te0[k] for k in range(_NK)]
    gam = [gb_v[pl.ds(16 * k, 16)] for k in range(_NK)]
    bet = [gb_v[pl.ds(_D + 16 * k, 16)] for k in range(_NK)]

    @pl.loop(0, _N_CHUNK)
    def _chunk(c):
        base = wbase + c * _CHUNK
        pltpu.sync_copy(ids_hbm.at[pl.ds(base, _CHUNK)], idx_v)
        pltpu.sync_copy(tt_hbm.at[pl.ds(base, _CHUNK)], tok_v)
        pltpu.async_copy(wemb_hbm.at[idx_v], rows_v, sem).wait()
        s0 = lax.rem(c * _CHUNK, _S)

        @pl.loop(0, _CHUNK, unroll=2)
        def _row(i):
            tf = jnp.float32(
                plsc.load_gather(tok_v, [jnp.full((16,), i, jnp.int32)]))
            pbase = (s0 + i) * _D
            x = [None] * _NK
            acc = jnp.zeros((16,), jnp.float32)
            accsq = jnp.zeros((16,), jnp.float32)
            for k in range(_NK):
                xk = (rows_v[i, pl.ds(16 * k, 16)]
                      + pos_v[pl.ds(pbase + 16 * k, 16)]
                      + (te0[k] + tf * dte[k]))
                x[k] = xk
                acc = acc + xk
                accsq = accsq + xk * xk
            mean = jnp.sum(acc) * (1.0 / _D)
            var = jnp.sum(accsq) * (1.0 / _D) - mean * mean
            meanv = jnp.full((16,), mean, jnp.float32)
            rstdv = _rsqrt(jnp.full((16,), var + _EPS, jnp.float32))
            for k in range(_NK):
                y = (x[k] - meanv) * rstdv * gam[k] + bet[k]
                rows_v[i, pl.ds(16 * k, 16)] = y

        pltpu.sync_copy(rows_v, out_hbm.at[pl.ds(base, _CHUNK)])


@jax.jit
def kernel(input_ids, token_type_ids, word_emb, pos_emb, tok_type_emb, gamma,
           beta):
    ids = input_ids.reshape(_N_TOK)
    tts = token_type_ids.reshape(_N_TOK)
    pos_flat = pos_emb.reshape(_S * _D)
    te_flat = tok_type_emb.reshape(2 * _D)
    mesh = plsc.VectorSubcoreMesh(core_axis_name="c", subcore_axis_name="s")
    run = functools.partial(
        pl.kernel,
        out_type=jax.ShapeDtypeStruct((_N_TOK, _D), jnp.float32),
        mesh=mesh,
        scratch_types=[
            pltpu.VMEM((_CHUNK,), jnp.int32),
            pltpu.VMEM((_CHUNK,), jnp.int32),
            pltpu.VMEM((_CHUNK, _D), jnp.float32),
            pltpu.VMEM((_S * _D,), jnp.float32),
            pltpu.VMEM((2 * _D,), jnp.float32),
            pltpu.VMEM((2 * _D,), jnp.float32),
            pltpu.SemaphoreType.DMA,
        ],
        compiler_params=pltpu.CompilerParams(needs_layout_passes=False),
    )(_body)
    return run(ids, tts, word_emb, pos_flat, te_flat, gamma, beta)
